# Initial kernel scaffold; baseline (speedup 1.0000x reference)
import jax
import jax.numpy as jnp
from jax.experimental import pallas as pl


def kernel(inputs, table, dense_w, dense_b):
    raise NotImplementedError("write your pallas kernel here")


# speedup vs baseline: 153.6835x; 153.6835x over previous
import jax
import jax.numpy as jnp
from jax import lax
from jax.experimental import pallas as pl
from jax.experimental.pallas import tpu as pltpu
from jax.experimental.pallas import tpu_sc as plsc

VOCAB = 10001
EMBED = 16
SEQ = 200
BATCH = 16384
VP = 10016
NC, NS, L = 2, 16, 16
NW = NC * NS
RPW = BATCH // NW
IPW = RPW * SEQ
PAIRS = RPW // 2


def _tw_body(table_ref, w_ref, out_ref):
    out_ref[...] = jnp.dot(
        table_ref[...], w_ref[...], preferred_element_type=jnp.float32
    ) * (1.0 / SEQ)


def _pool_body(tw_hbm, idx_hbm, bias_hbm, out_hbm, tw_v, idx_v, bias_v, out_v):
    wid = lax.axis_index("s") * NC + lax.axis_index("c")
    pltpu.sync_copy(tw_hbm, tw_v)
    pltpu.sync_copy(bias_hbm, bias_v)
    pltpu.sync_copy(idx_hbm.at[pl.ds(wid * IPW, IPW)], idx_v)
    bias = bias_v[pl.ds(0, L)][0]
    lane = lax.broadcasted_iota(jnp.int32, (L,), 0)
    first8 = lane < 8
    zero = jnp.zeros((L,), jnp.float32)

    def pair(p, carry):
        off = p * (2 * SEQ)
        accA = zero
        for j in range(12):
            inds = idx_v[pl.ds(off + j * L, L)]
            accA = accA + plsc.load_gather(tw_v, [inds])
        v = plsc.load_gather(tw_v, [idx_v[pl.ds(off + 12 * L, L)]])
        accA = accA + jnp.where(first8, v, zero)
        accB = jnp.where(first8, zero, v)
        for j in range(13, 25):
            inds = idx_v[pl.ds(off + j * L, L)]
            accB = accB + plsc.load_gather(tw_v, [inds])
        sA = jnp.sum(accA) + bias
        sB = jnp.sum(accB) + bias
        vals = jnp.where(lane < 1, sA, sB)
        plsc.store_scatter(out_v, [2 * p + lane], vals, mask=lane < 2)
        return carry

    lax.fori_loop(0, PAIRS, pair, 0)
    pltpu.sync_copy(out_v, out_hbm.at[pl.ds(wid * RPW, RPW)])


def kernel(inputs, table, dense_w, dense_b):
    idx = inputs.astype(jnp.int32).reshape(-1)
    table_p = jnp.pad(table, ((0, VP - table.shape[0]), (0, 0)))
    tw = pl.pallas_call(
        _tw_body,
        out_shape=jax.ShapeDtypeStruct((VP, 1), jnp.float32),
    )(table_p, dense_w).reshape(VP)
    bias16 = jnp.broadcast_to(dense_b.astype(jnp.float32), (L,))

    pool = pl.kernel(
        _pool_body,
        out_type=jax.ShapeDtypeStruct((BATCH,), jnp.float32),
        mesh=plsc.VectorSubcoreMesh(core_axis_name="c", subcore_axis_name="s"),
        scratch_types=[
            pltpu.VMEM((VP,), jnp.float32),
            pltpu.VMEM((IPW,), jnp.int32),
            pltpu.VMEM((L,), jnp.float32),
            pltpu.VMEM((RPW,), jnp.float32),
        ],
        compiler_params=pltpu.CompilerParams(needs_layout_passes=False),
    )
    out = pool(tw, idx, bias16)
    return out.reshape(BATCH, 1)


# speedup vs baseline: 176.8391x; 1.1507x over previous
import jax
import jax.numpy as jnp
from jax import lax
from jax.experimental import pallas as pl
from jax.experimental.pallas import tpu as pltpu
from jax.experimental.pallas import tpu_sc as plsc

VOCAB = 10001
EMBED = 16
SEQ = 200
BATCH = 16384
NC, NS, L = 2, 16, 16
NW = NC * NS
RPW = BATCH // NW
IPW = RPW * SEQ
PAIRS = RPW // 2


def _tw_body(table_ref, w_ref, b_ref, out_ref):
    w = w_ref[...]
    out_ref[...] = (jnp.sum(table_ref[...] * w, axis=1) + b_ref[0]) * (1.0 / SEQ)


def _pool_body(tw_hbm, idx_hbm, out_hbm, tw_v, idx_v, out_v):
    wid = lax.axis_index("s") * NC + lax.axis_index("c")
    pltpu.sync_copy(tw_hbm, tw_v)
    pltpu.sync_copy(idx_hbm.at[pl.ds(wid * IPW, IPW)], idx_v)
    lane = lax.broadcasted_iota(jnp.int32, (L,), 0)
    first8 = lane < 8
    zero = jnp.zeros((L,), jnp.float32)

    def pair(p, carry):
        off = p * (2 * SEQ)
        accA = zero
        for j in range(12):
            inds = idx_v[pl.ds(off + j * L, L)]
            accA = accA + plsc.load_gather(tw_v, [inds])
        v = plsc.load_gather(tw_v, [idx_v[pl.ds(off + 12 * L, L)]])
        accA = accA + jnp.where(first8, v, zero)
        accB = jnp.where(first8, zero, v)
        for j in range(13, 25):
            inds = idx_v[pl.ds(off + j * L, L)]
            accB = accB + plsc.load_gather(tw_v, [inds])
        sA = jnp.sum(accA)
        sB = jnp.sum(accB)
        vals = jnp.where(lane < 1, sA, sB)
        plsc.store_scatter(out_v, [2 * p + lane], vals, mask=lane < 2)
        return carry

    lax.fori_loop(0, PAIRS, pair, 0)
    pltpu.sync_copy(out_v, out_hbm.at[pl.ds(wid * RPW, RPW)])


def kernel(inputs, table, dense_w, dense_b):
    idx = inputs.astype(jnp.int32).reshape(-1)
    w_row = dense_w.reshape(1, EMBED)
    tw = pl.pallas_call(
        _tw_body,
        out_shape=jax.ShapeDtypeStruct((VOCAB,), jnp.float32),
    )(table, w_row, dense_b.astype(jnp.float32))

    pool = pl.kernel(
        _pool_body,
        out_type=jax.ShapeDtypeStruct((BATCH,), jnp.float32),
        mesh=plsc.VectorSubcoreMesh(core_axis_name="c", subcore_axis_name="s"),
        scratch_types=[
            pltpu.VMEM((VOCAB,), jnp.float32),
            pltpu.VMEM((IPW,), jnp.int32),
            pltpu.VMEM((RPW,), jnp.float32),
        ],
        compiler_params=pltpu.CompilerParams(needs_layout_passes=False),
    )
    out = pool(tw, idx)
    return out.reshape(BATCH, 1)
